# baseline (speedup 1.0000x reference)
import functools

import jax
import jax.numpy as jnp
from jax import lax
from jax.experimental import pallas as pl
from jax.experimental.pallas import tpu as pltpu
from jax.experimental.pallas import tpu_sc as plsc

_B = 4096
_S = 200
_D = 64
_NW = 32
_DPW = _D // _NW

_STC = 120
_SSC = _S - _STC
_R = 8
_NSB = _SSC // _R
_SB = 8
_NV = _B // 16
_GU = 16


def _tc_body(x_ref, o_ref):
    @pl.when(pl.program_id(0) == 0)
    def _init():
        o_ref[...] = jnp.zeros_like(o_ref)

    o_ref[...] += jnp.sum(x_ref[...], axis=0)


def _tc_sum(x3):
    return pl.pallas_call(
        _tc_body,
        grid=(_STC // _SB,),
        in_specs=[pl.BlockSpec((_SB, _D, _B), lambda i: (i, 0, 0))],
        out_specs=pl.BlockSpec((_D, _B), lambda i: (0, 0)),
        out_shape=jax.ShapeDtypeStruct((_D, _B), jnp.float32),
    )(x3)


def _sc_body(x_hbm, out_hbm, buf0, buf1, acc, sem0, sem1):
    cid = lax.axis_index("c")
    sid = lax.axis_index("s")
    w = sid * 2 + cid

    bufs = (buf0, buf1)
    sems = (sem0, sem1)

    def start(d, sb, slot):
        s0 = _STC + sb * _R
        return pltpu.async_copy(
            x_hbm.at[pl.ds(s0, _R), pl.ds(d, 1), :], bufs[slot], sems[slot]
        )

    def wait(slot):
        pltpu.make_async_copy(
            x_hbm.at[pl.ds(0, _R), pl.ds(0, 1), :], bufs[slot], sems[slot]
        ).wait()

    def accumulate(slot):
        buf = bufs[slot]

        def body(i2, _):
            base = i2 * (_GU * 16)
            for g in range(_GU):
                sl = pl.ds(base + g * 16, 16)
                t = []
                for r in range(0, _R, 2):
                    t.append(buf[r, 0, sl] + buf[r + 1, 0, sl])
                while len(t) > 1:
                    t = [t[i] + t[i + 1] for i in range(0, len(t) - 1, 2)] + (
                        [t[-1]] if len(t) % 2 else []
                    )
                plsc.addupdate(acc.at[0, sl], t[0])
            return _

        lax.fori_loop(0, _NV // _GU, body, 0)

    zero = jnp.zeros((16,), jnp.float32)

    for dd in range(_DPW):
        d = w * _DPW + dd

        def zbody(i2, _):
            for g in range(_GU):
                acc[0, pl.ds(i2 * (_GU * 16) + g * 16, 16)] = zero
            return _

        lax.fori_loop(0, _NV // _GU, zbody, 0)

        start(d, 0, 0)
        start(d, 1, 1)

        def outer(j2, _):
            sb = j2 * 2
            wait(0)
            accumulate(0)
            start(d, sb + 2, 0)
            wait(1)
            accumulate(1)
            start(d, sb + 3, 1)
            return _

        lax.fori_loop(0, _NSB // 2 - 1, outer, 0)
        wait(0)
        accumulate(0)
        wait(1)
        accumulate(1)
        pltpu.sync_copy(acc, out_hbm.at[pl.ds(d, 1), :])


def _sc_sum(x3):
    mesh = plsc.VectorSubcoreMesh(core_axis_name="c", subcore_axis_name="s")
    f = functools.partial(
        pl.kernel,
        mesh=mesh,
        out_type=jax.ShapeDtypeStruct((_D, _B), jnp.float32),
        scratch_types=[
            pltpu.VMEM((_R, 1, _B), jnp.float32),
            pltpu.VMEM((_R, 1, _B), jnp.float32),
            pltpu.VMEM((1, _B), jnp.float32),
            pltpu.SemaphoreType.DMA,
            pltpu.SemaphoreType.DMA,
        ],
    )(_sc_body)
    return f(x3)


def kernel(inputs):
    x3 = jnp.transpose(inputs, (1, 2, 0))
    out_sc = _sc_sum(x3)
    out_tc = _tc_sum(x3)
    out_t = out_tc + out_sc
    return jnp.transpose(out_t, (1, 0))


# speedup vs baseline: 1.0405x; 1.0405x over previous
import functools

import jax
import jax.numpy as jnp
from jax import lax
from jax.experimental import pallas as pl
from jax.experimental.pallas import tpu as pltpu
from jax.experimental.pallas import tpu_sc as plsc

_B = 4096
_S = 200
_D = 64
_NW = 32
_DPW = _D // _NW

_STC = 168
_SSC = _S - _STC
_R = 8
_NSB = _SSC // _R
_SB = 8
_NV = _B // 16
_GU = 16


def _tc_body(x_ref, o_ref):
    @pl.when(pl.program_id(0) == 0)
    def _init():
        o_ref[...] = jnp.zeros_like(o_ref)

    o_ref[...] += jnp.sum(x_ref[...], axis=0)


def _tc_sum(x3):
    return pl.pallas_call(
        _tc_body,
        grid=(_STC // _SB,),
        in_specs=[pl.BlockSpec((_SB, _D, _B), lambda i: (i, 0, 0))],
        out_specs=pl.BlockSpec((_D, _B), lambda i: (0, 0)),
        out_shape=jax.ShapeDtypeStruct((_D, _B), jnp.float32),
    )(x3)


def _sc_body(x_hbm, out_hbm, buf0, buf1, acc, sem0, sem1):
    cid = lax.axis_index("c")
    sid = lax.axis_index("s")
    w = sid * 2 + cid

    bufs = (buf0, buf1)
    sems = (sem0, sem1)

    def start(d, sb, slot):
        s0 = _STC + sb * _R
        return pltpu.async_copy(
            x_hbm.at[pl.ds(s0, _R), pl.ds(d, 1), :], bufs[slot], sems[slot]
        )

    def wait(slot):
        pltpu.make_async_copy(
            x_hbm.at[pl.ds(0, _R), pl.ds(0, 1), :], bufs[slot], sems[slot]
        ).wait()

    def accumulate(slot):
        buf = bufs[slot]

        def body(i2, _):
            base = i2 * (_GU * 16)
            for g in range(_GU):
                sl = pl.ds(base + g * 16, 16)
                t = []
                for r in range(0, _R, 2):
                    t.append(buf[r, 0, sl] + buf[r + 1, 0, sl])
                while len(t) > 1:
                    t = [t[i] + t[i + 1] for i in range(0, len(t) - 1, 2)] + (
                        [t[-1]] if len(t) % 2 else []
                    )
                plsc.addupdate(acc.at[0, sl], t[0])
            return _

        lax.fori_loop(0, _NV // _GU, body, 0)

    zero = jnp.zeros((16,), jnp.float32)

    for dd in range(_DPW):
        d = w * _DPW + dd

        def zbody(i2, _):
            for g in range(_GU):
                acc[0, pl.ds(i2 * (_GU * 16) + g * 16, 16)] = zero
            return _

        lax.fori_loop(0, _NV // _GU, zbody, 0)

        start(d, 0, 0)
        start(d, 1, 1)

        def outer(j2, _):
            sb = j2 * 2
            wait(0)
            accumulate(0)
            start(d, sb + 2, 0)
            wait(1)
            accumulate(1)
            start(d, sb + 3, 1)
            return _

        lax.fori_loop(0, _NSB // 2 - 1, outer, 0)
        wait(0)
        accumulate(0)
        wait(1)
        accumulate(1)
        pltpu.sync_copy(acc, out_hbm.at[pl.ds(d, 1), :])


def _sc_sum(x3):
    mesh = plsc.VectorSubcoreMesh(core_axis_name="c", subcore_axis_name="s")
    f = functools.partial(
        pl.kernel,
        mesh=mesh,
        out_type=jax.ShapeDtypeStruct((_D, _B), jnp.float32),
        scratch_types=[
            pltpu.VMEM((_R, 1, _B), jnp.float32),
            pltpu.VMEM((_R, 1, _B), jnp.float32),
            pltpu.VMEM((1, _B), jnp.float32),
            pltpu.SemaphoreType.DMA,
            pltpu.SemaphoreType.DMA,
        ],
    )(_sc_body)
    return f(x3)


def kernel(inputs):
    x3 = jnp.transpose(inputs, (1, 2, 0))
    out_sc = _sc_sum(x3)
    out_tc = _tc_sum(x3)
    out_t = out_tc + out_sc
    return jnp.transpose(out_t, (1, 0))


# speedup vs baseline: 1.3692x; 1.3159x over previous
import jax
import jax.numpy as jnp
from jax.experimental import pallas as pl

_B = 4096
_S = 200
_D = 64
_SB = 8


def _tc_body(x_ref, o_ref):
    @pl.when(pl.program_id(0) == 0)
    def _init():
        o_ref[...] = jnp.zeros_like(o_ref)

    o_ref[...] += jnp.sum(x_ref[...], axis=0)


def kernel(inputs):
    x3 = jnp.transpose(inputs, (1, 2, 0))
    out_t = pl.pallas_call(
        _tc_body,
        grid=(_S // _SB,),
        in_specs=[pl.BlockSpec((_SB, _D, _B), lambda i: (i, 0, 0))],
        out_specs=pl.BlockSpec((_D, _B), lambda i: (0, 0)),
        out_shape=jax.ShapeDtypeStruct((_D, _B), jnp.float32),
    )(x3)
    return jnp.transpose(out_t, (1, 0))
